# Initial kernel scaffold; baseline (speedup 1.0000x reference)
import functools

import jax
import jax.numpy as jnp
from jax import lax
from jax.experimental import pallas as pl
from jax.experimental.pallas import tpu as pltpu
from jax.experimental.pallas import tpu_sc as plsc

NU = 50000
NI = 50000
NB = 20000
EMB = 32
E_UI = 800000
E_BI = 640000
BATCH = 4096

NC, NS = 2, 16
ND = 50048
NBP = 20480
C = 2000

f32 = jnp.float32
i32 = jnp.int32

_MESH = plsc.VectorSubcoreMesh(core_axis_name="c", subcore_axis_name="s")


def _deg_body(ui_u, ui_i, bi_b, ones_hbm, zeros_hbm,
              deg_u, deg_i, bsz,
              idx_v, ones_v, acc_deg, acc_bs):
    core = lax.axis_index("c")
    sub = lax.axis_index("s")
    dpt = ND // NS
    bpt = NBP // NS
    d0 = sub * dpt
    b0 = sub * bpt
    pltpu.sync_copy(zeros_hbm.at[pl.ds(d0, dpt)], acc_deg.at[pl.ds(d0, dpt)])
    pltpu.sync_copy(zeros_hbm.at[pl.ds(b0, bpt)], acc_bs.at[pl.ds(b0, bpt)])
    pltpu.sync_copy(ones_hbm, ones_v)
    plsc.subcore_barrier()

    def count(idx_hbm, n_per_tile, acc):
        base = sub * n_per_tile

        def body(i, carry):
            pltpu.sync_copy(idx_hbm.at[pl.ds(base + i * C, C)], idx_v)
            pltpu.sync_copy(ones_v, acc.at[idx_v], add=True)
            return carry

        lax.fori_loop(0, n_per_tile // C, body, 0)

    @pl.when(core == 0)
    def _():
        count(ui_u, E_UI // NS, acc_deg)
        count(bi_b, E_BI // NS, acc_bs)

    @pl.when(core == 1)
    def _():
        count(ui_i, E_UI // NS, acc_deg)

    plsc.subcore_barrier()

    @pl.when(core == 0)
    def _():
        pltpu.sync_copy(acc_deg.at[pl.ds(d0, dpt)], deg_u.at[pl.ds(d0, dpt)])
        pltpu.sync_copy(acc_bs.at[pl.ds(b0, bpt)], bsz.at[pl.ds(b0, bpt)])

    @pl.when(core == 1)
    def _():
        pltpu.sync_copy(acc_deg.at[pl.ds(d0, dpt)], deg_i.at[pl.ds(d0, dpt)])


_deg_call = functools.partial(
    pl.kernel,
    out_type=(
        jax.ShapeDtypeStruct((ND,), f32),
        jax.ShapeDtypeStruct((ND,), f32),
        jax.ShapeDtypeStruct((NBP,), f32),
    ),
    mesh=_MESH,
    scratch_types=[
        pltpu.VMEM((C,), i32),
        pltpu.VMEM((C,), f32),
        pltpu.VMEM_SHARED((ND,), f32),
        pltpu.VMEM_SHARED((NBP,), f32),
    ],
)(_deg_body)


def _spmm_body(gu, gi, ui_u, ui_i, zeros_hbm,
               hu, hi,
               src_v, dst_v, rows_v, sem, acc):
    core = lax.axis_index("c")
    sub = lax.axis_index("s")
    rpt = NU // NS
    r0 = sub * rpt
    pltpu.sync_copy(zeros_hbm.at[pl.ds(r0, rpt)], acc.at[pl.ds(r0, rpt)])
    plsc.subcore_barrier()

    ept = E_UI // NS
    base = sub * ept

    def layer(src_idx, table, dst_idx):
        def body(i, carry):
            off = base + i * C
            pltpu.sync_copy(src_idx.at[pl.ds(off, C)], src_v)
            pltpu.sync_copy(dst_idx.at[pl.ds(off, C)], dst_v)
            pltpu.async_copy(table.at[src_v], rows_v, sem).wait()
            pltpu.sync_copy(rows_v, acc.at[dst_v], add=True)
            return carry

        lax.fori_loop(0, ept // C, body, 0)

    @pl.when(core == 0)
    def _():
        layer(ui_i, gi, ui_u)

    @pl.when(core == 1)
    def _():
        layer(ui_u, gu, ui_i)

    plsc.subcore_barrier()

    @pl.when(core == 0)
    def _():
        pltpu.sync_copy(acc.at[pl.ds(r0, rpt)], hu.at[pl.ds(r0, rpt)])

    @pl.when(core == 1)
    def _():
        pltpu.sync_copy(acc.at[pl.ds(r0, rpt)], hi.at[pl.ds(r0, rpt)])


_spmm_call = functools.partial(
    pl.kernel,
    out_type=(
        jax.ShapeDtypeStruct((NU, EMB), f32),
        jax.ShapeDtypeStruct((NI, EMB), f32),
    ),
    mesh=_MESH,
    scratch_types=[
        pltpu.VMEM((C,), i32),
        pltpu.VMEM((C,), i32),
        pltpu.VMEM((C, EMB), f32),
        pltpu.SemaphoreType.DMA,
        pltpu.VMEM_SHARED((NU, EMB), f32),
    ],
)(_spmm_body)


def _bi_body(ai, bi_b, bi_i, zeros_hbm,
             hb,
             src_v, dst_v, rows_v, sem, acc):
    core = lax.axis_index("c")
    sub = lax.axis_index("s")
    rpt = NB // NS
    r0 = sub * rpt
    pltpu.sync_copy(zeros_hbm.at[pl.ds(r0, rpt)], acc.at[pl.ds(r0, rpt)])
    plsc.subcore_barrier()

    ept = E_BI // (NC * NS)
    base = (core * NS + sub) * ept

    def body(i, carry):
        off = base + i * C
        pltpu.sync_copy(bi_i.at[pl.ds(off, C)], src_v)
        pltpu.sync_copy(bi_b.at[pl.ds(off, C)], dst_v)
        pltpu.async_copy(ai.at[src_v], rows_v, sem).wait()
        pltpu.sync_copy(rows_v, acc.at[dst_v], add=True)
        return carry

    lax.fori_loop(0, ept // C, body, 0)
    plsc.subcore_barrier()
    pltpu.sync_copy(acc.at[pl.ds(r0, rpt)], hb.at[pl.ds(core * NB + r0, rpt)])


_bi_call = functools.partial(
    pl.kernel,
    out_type=jax.ShapeDtypeStruct((NC * NB, EMB), f32),
    mesh=_MESH,
    scratch_types=[
        pltpu.VMEM((C,), i32),
        pltpu.VMEM((C,), i32),
        pltpu.VMEM((C, EMB), f32),
        pltpu.SemaphoreType.DMA,
        pltpu.VMEM_SHARED((NB, EMB), f32),
    ],
)(_bi_body)


UPW = BATCH // (NC * NS)
BPW = 2 * BATCH // (NC * NS)


def _lookup_body(au, brep, uidx, bidx,
                 ue, be,
                 iu_v, ib_v, ru_v, rb_v, sem):
    core = lax.axis_index("c")
    sub = lax.axis_index("s")
    wid = core * NS + sub
    u0 = wid * UPW
    pltpu.sync_copy(uidx.at[pl.ds(u0, UPW)], iu_v)
    pltpu.async_copy(au.at[iu_v], ru_v, sem).wait()
    pltpu.sync_copy(ru_v, ue.at[pl.ds(u0, UPW)])
    b0 = wid * BPW
    pltpu.sync_copy(bidx.at[pl.ds(b0, BPW)], ib_v)
    pltpu.async_copy(brep.at[ib_v], rb_v, sem).wait()
    pltpu.sync_copy(rb_v, be.at[pl.ds(b0, BPW)])


_lookup_call = functools.partial(
    pl.kernel,
    out_type=(
        jax.ShapeDtypeStruct((BATCH, EMB), f32),
        jax.ShapeDtypeStruct((2 * BATCH, EMB), f32),
    ),
    mesh=_MESH,
    scratch_types=[
        pltpu.VMEM((UPW,), i32),
        pltpu.VMEM((BPW,), i32),
        pltpu.VMEM((UPW, EMB), f32),
        pltpu.VMEM((BPW, EMB), f32),
        pltpu.SemaphoreType.DMA,
    ],
)(_lookup_body)


def _loss_body(u_ref, pos_ref, neg_ref, out_ref):
    u = u_ref[...]
    x = jnp.sum(u * (neg_ref[...] - pos_ref[...]), axis=1)
    sp = jnp.maximum(x, 0.0) + jnp.log(1.0 + jnp.exp(-jnp.abs(x)))
    out_ref[0, 0] = jnp.mean(sp)


def _loss_call(ue, pos, neg):
    return pl.pallas_call(
        _loss_body,
        out_shape=jax.ShapeDtypeStruct((1, 1), f32),
    )(ue, pos, neg)


def kernel(users_feature, items_feature, bundles_feature,
           ui_u, ui_i, bi_b, bi_i, users, bundles):
    ui_u = ui_u.astype(i32)
    ui_i = ui_i.astype(i32)
    bi_b = bi_b.astype(i32)
    bi_i = bi_i.astype(i32)

    ones_c = jnp.ones((C,), f32)
    zeros_1d = jnp.zeros((ND,), f32)
    zeros_2d = jnp.zeros((NU, EMB), f32)

    deg_u, deg_i, bsz = _deg_call(ui_u, ui_i, bi_b, ones_c, zeros_1d)
    ru = 1.0 / (jnp.sqrt(deg_u[:NU]) + 1e-8)
    ri = 1.0 / (jnp.sqrt(deg_i[:NI]) + 1e-8)
    binv = 1.0 / (bsz[:NB] + 1e-8)

    g0u = users_feature * ru[:, None]
    g0i = items_feature * ri[:, None]
    h1u, h1i = _spmm_call(g0u, g0i, ui_u, ui_i, zeros_2d)
    f1u = h1u * ru[:, None]
    f1i = h1i * ri[:, None]
    h2u, h2i = _spmm_call(f1u * ru[:, None], f1i * ri[:, None],
                          ui_u, ui_i, zeros_2d)
    f2u = h2u * ru[:, None]
    f2i = h2i * ri[:, None]

    au = (users_feature + f1u + f2u) / 3.0
    ai = (items_feature + f1i + f2i) / 3.0

    hb = _bi_call(ai, bi_b, bi_i, zeros_2d)
    brep = (hb[:NB] + hb[NB:]) * binv[:, None]

    ue, be = _lookup_call(au, brep,
                          users.reshape(-1).astype(i32),
                          bundles.reshape(-1).astype(i32))
    be = be.reshape(BATCH, 2, EMB)
    loss = _loss_call(ue, be[:, 0, :], be[:, 1, :])
    return (loss[0, 0], jnp.zeros(1, f32))


# speedup vs baseline: 33.3804x; 33.3804x over previous
import functools

import jax
import jax.numpy as jnp
from jax import lax
from jax.experimental import pallas as pl
from jax.experimental.pallas import tpu as pltpu
from jax.experimental.pallas import tpu_sc as plsc

NU = 50000
NI = 50000
NB = 20000
EMB = 32
E_UI = 800000
E_BI = 640000
BATCH = 4096

NC, NS = 2, 16
ND = 51200
NBP = 20480
NUP = 51200
NBA = 20480
C = 2000

f32 = jnp.float32
i32 = jnp.int32

_MESH = plsc.VectorSubcoreMesh(core_axis_name="c", subcore_axis_name="s")
_SC_PARAMS = pltpu.CompilerParams(use_tc_tiling_on_sc=False)


def _deg_body(ui_u, ui_i, bi_b, ones_hbm,
              deg_u, deg_i, bsz,
              idx_v, ones_v, zbuf, acc_deg, acc_bs):
    core = lax.axis_index("c")
    sub = lax.axis_index("s")
    dpt = ND // NS
    bpt = NBP // NS
    d0 = sub * dpt
    b0 = sub * bpt

    def fill(i, carry):
        zbuf[pl.ds(i * 16, 16)] = jnp.zeros((16,), f32)
        return carry

    lax.fori_loop(0, dpt // 16, fill, 0)
    pltpu.sync_copy(zbuf, acc_deg.at[pl.ds(d0, dpt)])
    pltpu.sync_copy(zbuf.at[pl.ds(0, bpt)], acc_bs.at[pl.ds(b0, bpt)])
    pltpu.sync_copy(ones_hbm, ones_v)
    plsc.subcore_barrier()

    def count(idx_hbm, n_per_tile, acc):
        base = sub * n_per_tile

        def body(i, carry):
            pltpu.sync_copy(idx_hbm.at[pl.ds(base + i * C, C)], idx_v)
            pltpu.sync_copy(ones_v, acc.at[idx_v], add=True)
            return carry

        lax.fori_loop(0, n_per_tile // C, body, 0)

    @pl.when(core == 0)
    def _():
        count(ui_u, E_UI // NS, acc_deg)
        count(bi_b, E_BI // NS, acc_bs)

    @pl.when(core == 1)
    def _():
        count(ui_i, E_UI // NS, acc_deg)

    plsc.subcore_barrier()

    @pl.when(core == 0)
    def _():
        pltpu.sync_copy(acc_deg.at[pl.ds(d0, dpt)], zbuf)
        pltpu.sync_copy(zbuf, deg_u.at[pl.ds(d0, dpt)])
        pltpu.sync_copy(acc_bs.at[pl.ds(b0, bpt)], zbuf.at[pl.ds(0, bpt)])
        pltpu.sync_copy(zbuf.at[pl.ds(0, bpt)], bsz.at[pl.ds(b0, bpt)])

    @pl.when(core == 1)
    def _():
        pltpu.sync_copy(acc_deg.at[pl.ds(d0, dpt)], zbuf)
        pltpu.sync_copy(zbuf, deg_i.at[pl.ds(d0, dpt)])


_deg_call = functools.partial(
    pl.kernel,
    out_type=(
        jax.ShapeDtypeStruct((ND,), f32),
        jax.ShapeDtypeStruct((ND,), f32),
        jax.ShapeDtypeStruct((NBP,), f32),
    ),
    mesh=_MESH,
    compiler_params=_SC_PARAMS,
    scratch_types=[
        pltpu.VMEM((C,), i32),
        pltpu.VMEM((C,), f32),
        pltpu.VMEM((ND // NS,), f32),
        pltpu.VMEM_SHARED((ND,), f32),
        pltpu.VMEM_SHARED((NBP,), f32),
    ],
)(_deg_body)


ZR = 640


def _fill_zeros(zb, width):
    def fill(i, carry):
        for w in range(width // 16):
            zb[i, pl.ds(w * 16, 16)] = jnp.zeros((16,), f32)
        return carry

    lax.fori_loop(0, ZR, fill, 0)


HEMB = EMB // 2


def _spmm_body(gu, gi, ui_u, ui_i,
               hu, hi,
               src_v, dst_v, rows_v, zb, sem, acc):
    core = lax.axis_index("c")
    sub = lax.axis_index("s")
    rpt = NUP // NS
    r0 = sub * rpt
    _fill_zeros(zb, HEMB)
    for k in range(rpt // ZR):
        pltpu.sync_copy(zb, acc.at[pl.ds(r0 + k * ZR, ZR)])
    plsc.subcore_barrier()

    ept = E_UI // NS
    base = sub * ept

    def layer(src_idx, table, dst_idx):
        def body(i, carry):
            off = base + i * C
            pltpu.sync_copy(src_idx.at[pl.ds(off, C)], src_v)
            pltpu.sync_copy(dst_idx.at[pl.ds(off, C)], dst_v)
            pltpu.async_copy(table.at[src_v], rows_v, sem).wait()
            pltpu.sync_copy(rows_v, acc.at[dst_v], add=True)
            return carry

        lax.fori_loop(0, ept // C, body, 0)

    @pl.when(core == 0)
    def _():
        layer(ui_i, gi, ui_u)

    @pl.when(core == 1)
    def _():
        layer(ui_u, gu, ui_i)

    plsc.subcore_barrier()

    def drain(h):
        for k in range(rpt // ZR):
            pltpu.sync_copy(acc.at[pl.ds(r0 + k * ZR, ZR)], zb)
            pltpu.sync_copy(zb, h.at[pl.ds(r0 + k * ZR, ZR)])

    @pl.when(core == 0)
    def _():
        drain(hu)

    @pl.when(core == 1)
    def _():
        drain(hi)


_spmm_call = functools.partial(
    pl.kernel,
    out_type=(
        jax.ShapeDtypeStruct((NUP, HEMB), f32),
        jax.ShapeDtypeStruct((NUP, HEMB), f32),
    ),
    mesh=_MESH,
    compiler_params=_SC_PARAMS,
    scratch_types=[
        pltpu.VMEM((C,), i32),
        pltpu.VMEM((C,), i32),
        pltpu.VMEM((C, HEMB), f32),
        pltpu.VMEM((ZR, HEMB), f32),
        pltpu.SemaphoreType.DMA,
        pltpu.VMEM_SHARED((NUP, HEMB), f32),
    ],
)(_spmm_body)


def _bi_body(ai, bi_b, bi_i,
             hb,
             src_v, dst_v, rows_v, zb, sem, acc):
    core = lax.axis_index("c")
    sub = lax.axis_index("s")
    rpt = NBA // NS
    r0 = sub * rpt
    _fill_zeros(zb, EMB)
    for k in range(rpt // ZR):
        pltpu.sync_copy(zb, acc.at[pl.ds(r0 + k * ZR, ZR)])
    plsc.subcore_barrier()

    ept = E_BI // (NC * NS)
    base = (core * NS + sub) * ept

    def body(i, carry):
        off = base + i * C
        pltpu.sync_copy(bi_i.at[pl.ds(off, C)], src_v)
        pltpu.sync_copy(bi_b.at[pl.ds(off, C)], dst_v)
        pltpu.async_copy(ai.at[src_v], rows_v, sem).wait()
        pltpu.sync_copy(rows_v, acc.at[dst_v], add=True)
        return carry

    lax.fori_loop(0, ept // C, body, 0)
    plsc.subcore_barrier()
    for k in range(rpt // ZR):
        pltpu.sync_copy(acc.at[pl.ds(r0 + k * ZR, ZR)], zb)
        pltpu.sync_copy(zb, hb.at[pl.ds(core * NBA + r0 + k * ZR, ZR)])


_bi_call = functools.partial(
    pl.kernel,
    out_type=jax.ShapeDtypeStruct((NC * NBA, EMB), f32),
    mesh=_MESH,
    compiler_params=_SC_PARAMS,
    scratch_types=[
        pltpu.VMEM((C,), i32),
        pltpu.VMEM((C,), i32),
        pltpu.VMEM((C, EMB), f32),
        pltpu.VMEM((ZR, EMB), f32),
        pltpu.SemaphoreType.DMA,
        pltpu.VMEM_SHARED((NBA, EMB), f32),
    ],
)(_bi_body)


UPW = BATCH // (NC * NS)
BPW = 2 * BATCH // (NC * NS)


def _lookup_body(au, brep, uidx, bidx,
                 ue, be,
                 iu_v, ib_v, ru_v, rb_v, sem):
    core = lax.axis_index("c")
    sub = lax.axis_index("s")
    wid = core * NS + sub
    u0 = wid * UPW
    pltpu.sync_copy(uidx.at[pl.ds(u0, UPW)], iu_v)
    pltpu.async_copy(au.at[iu_v], ru_v, sem).wait()
    pltpu.sync_copy(ru_v, ue.at[pl.ds(u0, UPW)])
    b0 = wid * BPW
    pltpu.sync_copy(bidx.at[pl.ds(b0, BPW)], ib_v)
    pltpu.async_copy(brep.at[ib_v], rb_v, sem).wait()
    pltpu.sync_copy(rb_v, be.at[pl.ds(b0, BPW)])


_lookup_call = functools.partial(
    pl.kernel,
    out_type=(
        jax.ShapeDtypeStruct((BATCH, EMB), f32),
        jax.ShapeDtypeStruct((2 * BATCH, EMB), f32),
    ),
    mesh=_MESH,
    compiler_params=_SC_PARAMS,
    scratch_types=[
        pltpu.VMEM((UPW,), i32),
        pltpu.VMEM((BPW,), i32),
        pltpu.VMEM((UPW, EMB), f32),
        pltpu.VMEM((BPW, EMB), f32),
        pltpu.SemaphoreType.DMA,
    ],
)(_lookup_body)


def _loss_body(u_ref, pos_ref, neg_ref, out_ref):
    u = u_ref[...]
    x = jnp.sum(u * (neg_ref[...] - pos_ref[...]), axis=1)
    sp = jnp.maximum(x, 0.0) + jnp.log(1.0 + jnp.exp(-jnp.abs(x)))
    out_ref[...] = jnp.broadcast_to(jnp.mean(sp), (1, 1))


def _loss_call(ue, pos, neg):
    return pl.pallas_call(
        _loss_body,
        out_shape=jax.ShapeDtypeStruct((1, 1), f32),
    )(ue, pos, neg)


def kernel(users_feature, items_feature, bundles_feature,
           ui_u, ui_i, bi_b, bi_i, users, bundles):
    ui_u = ui_u.astype(i32)
    ui_i = ui_i.astype(i32)
    bi_b = bi_b.astype(i32)
    bi_i = bi_i.astype(i32)

    ones_c = jnp.ones((C,), f32)

    deg_u, deg_i, bsz = _deg_call(ui_u, ui_i, bi_b, ones_c)
    ru = 1.0 / (jnp.sqrt(deg_u[:NU]) + 1e-8)
    ri = 1.0 / (jnp.sqrt(deg_i[:NI]) + 1e-8)
    binv = 1.0 / (bsz[:NB] + 1e-8)

    def spmm(gu, gi):
        parts = [_spmm_call(gu[:, w * HEMB:(w + 1) * HEMB],
                            gi[:, w * HEMB:(w + 1) * HEMB],
                            ui_u, ui_i)
                 for w in range(EMB // HEMB)]
        hu = jnp.concatenate([p[0][:NU] for p in parts], axis=1)
        hi = jnp.concatenate([p[1][:NI] for p in parts], axis=1)
        return hu, hi

    g0u = users_feature * ru[:, None]
    g0i = items_feature * ri[:, None]
    h1u, h1i = spmm(g0u, g0i)
    f1u = h1u * ru[:, None]
    f1i = h1i * ri[:, None]
    h2u, h2i = spmm(f1u * ru[:, None], f1i * ri[:, None])
    f2u = h2u * ru[:, None]
    f2i = h2i * ri[:, None]

    au = (users_feature + f1u + f2u) / 3.0
    ai = (items_feature + f1i + f2i) / 3.0

    hb = _bi_call(ai, bi_b, bi_i)
    brep = (hb[:NB] + hb[NBA:NBA + NB]) * binv[:, None]

    ue, be = _lookup_call(au, brep,
                          users.reshape(-1).astype(i32),
                          bundles.reshape(-1).astype(i32))
    be = be.reshape(BATCH, 2, EMB)
    loss = _loss_call(ue, be[:, 0, :], be[:, 1, :])
    return (loss[0, 0], jnp.zeros(1, f32))
